# baseline (speedup 1.0000x reference)
import functools

import jax
import jax.numpy as jnp
from jax import lax
from jax.experimental import pallas as pl
from jax.experimental.pallas import tpu as pltpu
from jax.experimental.pallas import tpu_sc as plsc

ALPHA = 3.0
K = 8
TR = 512
NBUF = 2


def _nodevec_body(x_ref, w1_ref, b1_ref, w2_ref, b2_ref, nv1_ref, nv2_ref):
    x = x_ref[0]
    nv1_ref[0] = jnp.tanh(ALPHA * (jnp.dot(x, w1_ref[...],
                                           preferred_element_type=jnp.float32)
                                   + b1_ref[...]))
    nv2_ref[0] = jnp.tanh(ALPHA * (jnp.dot(x, w2_ref[...],
                                           preferred_element_type=jnp.float32)
                                   + b2_ref[...]))


def _topk_body(nv1_ref, nv2_ref, val_ref, idx_ref, *, n_rows):
    r = pl.program_id(1)
    n1 = nv1_ref[0]
    n2 = nv2_ref[0]
    rows1 = nv1_ref[0, pl.ds(r * n_rows, n_rows), :]
    rows2 = nv2_ref[0, pl.ds(r * n_rows, n_rows), :]

    contract = (((1,), (1,)), ((), ()))
    s1 = jax.lax.dot_general(rows1, n2, contract,
                             preferred_element_type=jnp.float32)
    s2 = jax.lax.dot_general(rows2, n1, contract,
                             preferred_element_type=jnp.float32)
    adj = jnp.maximum(jnp.tanh(ALPHA * (s1 - s2)), 0.0)

    idx = jax.lax.broadcasted_iota(jnp.int32, adj.shape, 1).astype(jnp.float32)
    work = adj
    n_cols = float(adj.shape[1])
    vals = []
    idxs = []
    for k in range(K):
        m = jnp.max(work, axis=1, keepdims=True)
        mi = jnp.min(jnp.where(work == m, idx, n_cols), axis=1, keepdims=True)
        vals.append(m)
        idxs.append(mi)
        if k < K - 1:
            work = jnp.where(idx == mi, -1.0, work)

    val_ref[0] = jnp.concatenate(vals, axis=1)
    idx_ref[0] = jnp.concatenate(idxs, axis=1).astype(jnp.int32)


def _make_sc_scatter(R, N, rpw):
    mesh = plsc.VectorSubcoreMesh(core_axis_name="c", subcore_axis_name="s")

    @functools.partial(
        pl.kernel,
        out_type=jax.ShapeDtypeStruct((R, N), jnp.float32),
        mesh=mesh,
        scratch_types=[
            pltpu.VMEM((rpw * K + 16,), jnp.int32),
            pltpu.VMEM((rpw * K + 16,), jnp.float32),
            pltpu.VMEM((N,), jnp.float32),
            pltpu.VMEM((N,), jnp.float32),
            pltpu.SemaphoreType.DMA,
            pltpu.SemaphoreType.DMA,
        ],
        compiler_params=pltpu.CompilerParams(needs_layout_passes=False),
    )
    def sc_scatter(val_hbm, idx_hbm, out_hbm, idx_sl, val_sl, buf0, buf1,
                   sem0, sem1):
        wid = lax.axis_index("s") * 2 + lax.axis_index("c")
        base = wid * rpw
        bufs = (buf0, buf1)
        sems = (sem0, sem1)

        pltpu.sync_copy(idx_hbm.at[pl.ds(base * K, rpw * K)],
                        idx_sl.at[pl.ds(0, rpw * K)])
        pltpu.sync_copy(val_hbm.at[pl.ds(base * K, rpw * K)],
                        val_sl.at[pl.ds(0, rpw * K)])

        zero16 = jnp.zeros((16,), jnp.float32)
        lane = lax.broadcasted_iota(jnp.int32, (16,), 0)
        mask = lane < K

        def _zero_bufs(i, _):
            buf0[pl.ds(i * 16, 16)] = zero16
            buf1[pl.ds(i * 16, 16)] = zero16
            return _

        lax.fori_loop(0, N // 16, _zero_bufs, None)

        def _emit(row, b):
            iv = idx_sl[pl.ds(row * K, 16)]
            vv = val_sl[pl.ds(row * K, 16)]
            plsc.store_scatter(bufs[b], [iv], vv, mask=mask)
            return pltpu.async_copy(bufs[b], out_hbm.at[base + row], sems[b])

        for b in range(NBUF):
            _emit(b, b)

        def _step(g, _):
            row0 = g * NBUF
            for b in range(NBUF):
                row = row0 + b
                old = row - NBUF
                pltpu.make_async_copy(bufs[b], out_hbm.at[base + old],
                                      sems[b]).wait()
                oiv = idx_sl[pl.ds(old * K, 16)]
                plsc.store_scatter(bufs[b], [oiv], zero16, mask=mask)
                _emit(row, b)
            return _

        lax.fori_loop(1, rpw // NBUF, _step, None)

        for b in range(NBUF):
            row = rpw - NBUF + b
            pltpu.make_async_copy(bufs[b], out_hbm.at[base + row],
                                  sems[b]).wait()

    return sc_scatter


def kernel(X, W1, b1, W2, b2):
    B, N, FD = X.shape
    D = W1.shape[1]
    R = B * N

    nv1, nv2 = pl.pallas_call(
        _nodevec_body,
        grid=(B,),
        in_specs=[
            pl.BlockSpec((1, N, FD), lambda b: (b, 0, 0)),
            pl.BlockSpec((FD, D), lambda b: (0, 0)),
            pl.BlockSpec((D,), lambda b: (0,)),
            pl.BlockSpec((FD, D), lambda b: (0, 0)),
            pl.BlockSpec((D,), lambda b: (0,)),
        ],
        out_specs=[
            pl.BlockSpec((1, N, D), lambda b: (b, 0, 0)),
            pl.BlockSpec((1, N, D), lambda b: (b, 0, 0)),
        ],
        out_shape=[
            jax.ShapeDtypeStruct((B, N, D), jnp.float32),
            jax.ShapeDtypeStruct((B, N, D), jnp.float32),
        ],
    )(X, W1, b1, W2, b2)

    vals, idxs = pl.pallas_call(
        functools.partial(_topk_body, n_rows=TR),
        grid=(B, N // TR),
        in_specs=[
            pl.BlockSpec((1, N, D), lambda b, r: (b, 0, 0)),
            pl.BlockSpec((1, N, D), lambda b, r: (b, 0, 0)),
        ],
        out_specs=[
            pl.BlockSpec((1, TR, K), lambda b, r: (b, r, 0)),
            pl.BlockSpec((1, TR, K), lambda b, r: (b, r, 0)),
        ],
        out_shape=[
            jax.ShapeDtypeStruct((B, N, K), jnp.float32),
            jax.ShapeDtypeStruct((B, N, K), jnp.int32),
        ],
        compiler_params=pltpu.CompilerParams(
            dimension_semantics=("parallel", "arbitrary"),
        ),
    )(nv1, nv2)

    rpw = R // 32
    out = _make_sc_scatter(R, N, rpw)(vals.reshape(R * K), idxs.reshape(R * K))
    return out.reshape(B, N, N)


# speedup vs baseline: 1.1584x; 1.1584x over previous
import functools

import jax
import jax.numpy as jnp
from jax.experimental import pallas as pl
from jax.experimental.pallas import tpu as pltpu

ALPHA = 3.0
K = 8
TR = 512


def _nodevec_body(x_ref, w1_ref, b1_ref, w2_ref, b2_ref, nv1_ref, nv2_ref):
    x = x_ref[0]
    nv1_ref[0] = jnp.tanh(ALPHA * (jnp.dot(x, w1_ref[...],
                                           preferred_element_type=jnp.float32)
                                   + b1_ref[...]))
    nv2_ref[0] = jnp.tanh(ALPHA * (jnp.dot(x, w2_ref[...],
                                           preferred_element_type=jnp.float32)
                                   + b2_ref[...]))


def _adj_topk_body(nv1_ref, nv2_ref, out_ref, *, n_rows):
    r = pl.program_id(1)
    n1 = nv1_ref[0]
    n2 = nv2_ref[0]
    rows1 = nv1_ref[0, pl.ds(r * n_rows, n_rows), :]
    rows2 = nv2_ref[0, pl.ds(r * n_rows, n_rows), :]

    contract = (((1,), (1,)), ((), ()))
    s1 = jax.lax.dot_general(rows1, n2, contract,
                             preferred_element_type=jnp.float32)
    s2 = jax.lax.dot_general(rows2, n1, contract,
                             preferred_element_type=jnp.float32)
    adj = jnp.maximum(jnp.tanh(ALPHA * (s1 - s2)), 0.0)

    idx = jax.lax.broadcasted_iota(jnp.int32, adj.shape, 1).astype(jnp.float32)
    work = adj
    n_cols = float(adj.shape[1])
    for _ in range(K - 1):
        m = jnp.max(work, axis=1, keepdims=True)
        mi = jnp.min(jnp.where(work == m, idx, n_cols), axis=1, keepdims=True)
        work = jnp.where(idx == mi, -1.0, work)
    m = jnp.max(work, axis=1, keepdims=True)
    mi = jnp.min(jnp.where(work == m, idx, n_cols), axis=1, keepdims=True)

    out_ref[0] = jnp.where(jnp.logical_or(work < 0.0, idx == mi), adj, 0.0)


def kernel(X, W1, b1, W2, b2):
    B, N, FD = X.shape
    D = W1.shape[1]

    nv1, nv2 = pl.pallas_call(
        _nodevec_body,
        grid=(B,),
        in_specs=[
            pl.BlockSpec((1, N, FD), lambda b: (b, 0, 0)),
            pl.BlockSpec((FD, D), lambda b: (0, 0)),
            pl.BlockSpec((D,), lambda b: (0,)),
            pl.BlockSpec((FD, D), lambda b: (0, 0)),
            pl.BlockSpec((D,), lambda b: (0,)),
        ],
        out_specs=[
            pl.BlockSpec((1, N, D), lambda b: (b, 0, 0)),
            pl.BlockSpec((1, N, D), lambda b: (b, 0, 0)),
        ],
        out_shape=[
            jax.ShapeDtypeStruct((B, N, D), jnp.float32),
            jax.ShapeDtypeStruct((B, N, D), jnp.float32),
        ],
    )(X, W1, b1, W2, b2)

    out = pl.pallas_call(
        functools.partial(_adj_topk_body, n_rows=TR),
        grid=(B, N // TR),
        in_specs=[
            pl.BlockSpec((1, N, D), lambda b, r: (b, 0, 0)),
            pl.BlockSpec((1, N, D), lambda b, r: (b, 0, 0)),
        ],
        out_specs=pl.BlockSpec((1, TR, N), lambda b, r: (b, r, 0)),
        out_shape=jax.ShapeDtypeStruct((B, N, N), jnp.float32),
        compiler_params=pltpu.CompilerParams(
            dimension_semantics=("parallel", "arbitrary"),
            vmem_limit_bytes=100 * 1024 * 1024,
        ),
    )(nv1, nv2)
    return out
